# Initial kernel scaffold; baseline (speedup 1.0000x reference)
import jax
import jax.numpy as jnp
from jax.experimental import pallas as pl


def kernel(external_input, v, theta, lateral_weights, population_activity, homeo_bias):
    raise NotImplementedError("write your pallas kernel here")


# speedup vs baseline: 1.0116x; 1.0116x over previous
import jax
import jax.numpy as jnp
from jax.experimental import pallas as pl
from jax.experimental.pallas import tpu as pltpu

B, P = 32, 8192
BJ = 512
NJ = P // BJ
NCORES = 2
NJH = NJ // NCORES

DT = 1.0
TAU_MEM = 20.0
THETA0 = 1.0
TAU_THETA = 100.0
THETA_PLUS = 0.05
HOMEO_ETA = 0.01
TARGET_RATE = 0.05


def _body(ext_ref, v_ref, theta_ref, w_ref, pop_ref, homeo_ref,
          spikes_ref, vout_ref, thout_ref, total_ref, act_ref, homeo_out_ref):
    q = pl.program_id(0)
    j0 = pl.multiple_of(q * BJ, BJ)

    pop_row = pop_ref[...]
    w = w_ref[...]
    full = jnp.dot(pop_row, w, preferred_element_type=jnp.float32)

    d_blk = w_ref[pl.ds(j0, BJ), :]
    rows = jax.lax.broadcasted_iota(jnp.int32, (BJ, BJ), 0)
    cols = jax.lax.broadcasted_iota(jnp.int32, (BJ, BJ), 1)
    diag_row = jnp.sum(jnp.where(rows == cols, d_blk, 0.0), axis=0,
                       keepdims=True)
    pop_sub = pop_ref[:, pl.ds(j0, BJ)]
    lateral = -(full - pop_sub * diag_row)

    homeo_row = homeo_ref[...]
    total = ext_ref[...] + lateral + homeo_row
    v_old = v_ref[...]
    v_new = v_old + DT * (-v_old / TAU_MEM + total)
    theta = theta_ref[...]
    spikes = (v_new >= theta).astype(jnp.float32)
    vout_ref[...] = v_new * (1.0 - spikes)
    thout_ref[...] = theta + DT * (-(theta - THETA0) / TAU_THETA) + spikes * THETA_PLUS
    spikes_ref[...] = spikes
    total_ref[...] = total
    act = jnp.mean(spikes, axis=0, keepdims=True)
    act_ref[...] = act
    homeo_out_ref[...] = homeo_row + HOMEO_ETA * (TARGET_RATE - act)


def kernel(external_input, v, theta, lateral_weights, population_activity, homeo_bias):
    pop2 = population_activity.reshape(1, P)
    homeo2 = homeo_bias.reshape(1, P)

    bspec = pl.BlockSpec((B, BJ), lambda j: (0, j))
    rspec = pl.BlockSpec((1, BJ), lambda j: (0, j))

    out_shapes = (
        jax.ShapeDtypeStruct((B, P), jnp.float32),
        jax.ShapeDtypeStruct((B, P), jnp.float32),
        jax.ShapeDtypeStruct((B, P), jnp.float32),
        jax.ShapeDtypeStruct((B, P), jnp.float32),
        jax.ShapeDtypeStruct((1, P), jnp.float32),
        jax.ShapeDtypeStruct((1, P), jnp.float32),
    )

    spikes, v_out, theta_out, total, act, homeo_out = pl.pallas_call(
        _body,
        grid=(NJ,),
        in_specs=[
            bspec,
            bspec,
            bspec,
            pl.BlockSpec((P, BJ), lambda j: (0, j)),
            pl.BlockSpec((1, P), lambda j: (0, 0)),
            rspec,
        ],
        out_specs=[bspec, bspec, bspec, bspec, rspec, rspec],
        out_shape=out_shapes,
        compiler_params=pltpu.CompilerParams(
            dimension_semantics=("arbitrary",),
            vmem_limit_bytes=55 * 1024 * 1024,
        ),
    )(external_input, v, theta, lateral_weights, pop2, homeo2)

    return spikes, v_out, theta_out, total, act.reshape(P), homeo_out.reshape(P)
